# baseline (speedup 1.0000x reference)
import jax
import jax.numpy as jnp
from jax import lax
from jax.experimental import pallas as pl
from jax.experimental.pallas import tpu as pltpu
from jax.experimental.pallas import tpu_sc as plsc

N = 100000
D = 128
S = 400000
P = 2000
EC = 64000

NC = 2
NS = 16
NW = NC * NS

CH = 64
WPW = 12544
NCH_FULL = WPW // CH
NCH_LAST = (S - (NW - 1) * WPW) // CH
AR = 2048
FB = 80


def _sc_main(hQK, hV, mapper, batch):
    mesh = plsc.VectorSubcoreMesh(core_axis_name="c", subcore_axis_name="s",
                                  num_cores=NC, num_subcores=NS)

    def body(hqk_hbm, hv_hbm, map_hbm, bat_hbm,
             kk_hbm, v_hbm, parts_hbm,
             bufQ0, bufQ1, bufV0, bufV1, kkb0, kkb1,
             idx0, idx1, bat0, bat1, flq, flk, flc, fidx,
             accq, acck, accc,
             gsem0, gsem1, isem0, isem1, wk0, wk1, wv0, wv1):
        c = lax.axis_index("c")
        s = lax.axis_index("s")
        w = c * NS + s
        wbase = w * WPW
        nch = jnp.where(w == NW - 1, NCH_LAST, NCH_FULL)

        bufQ = (bufQ0, bufQ1)
        bufV = (bufV0, bufV1)
        kkb = (kkb0, kkb1)
        idxv = (idx0, idx1)
        batv = (bat0, bat1)
        gsem = (gsem0, gsem1)
        isem = (isem0, isem1)
        wk = (wk0, wk1)
        wv = (wv0, wv1)

        pltpu.sync_copy(map_hbm.at[pl.ds(wbase, CH)], idx0)
        pltpu.sync_copy(bat_hbm.at[pl.ds(wbase, CH)], bat0)
        pltpu.async_copy(hqk_hbm.at[idx0], bufQ0, gsem0)
        pltpu.async_copy(hv_hbm.at[idx0], bufV0, gsem0)
        pltpu.async_copy(map_hbm.at[pl.ds(wbase + CH, CH)], idx1, isem1)
        pltpu.async_copy(bat_hbm.at[pl.ds(wbase + CH, CH)], bat1, isem1)

        zero16 = jnp.zeros((16,), jnp.float32)

        def _zf(i, _):
            for k in range(8):
                flq[i, pl.ds(k * 16, 16)] = zero16
            return 0
        lax.fori_loop(0, FB, _zf, 0)
        for acc in (accq, acck, accc):
            pltpu.sync_copy(flq, acc.at[pl.ds(s * 128, FB)])
            pltpu.sync_copy(flq.at[pl.ds(0, 48)], acc.at[pl.ds(s * 128 + FB, 48)])
        psplat = jnp.full((16,), P, jnp.int32)
        for k in range(FB // 16):
            fidx[pl.ds(k * 16, 16)] = psplat
        plsc.subcore_barrier()

        iota16 = lax.iota(jnp.int32, 16)
        lane0 = iota16 == 0
        e0 = jnp.where(lane0, 1.0, 0.0).astype(jnp.float32)

        def flush_accs(nf, cur, accs):
            for k in range(8):
                flq[nf, pl.ds(k * 16, 16)] = accs[k]
            for k in range(8):
                flk[nf, pl.ds(k * 16, 16)] = accs[8 + k]
            flc[nf, pl.ds(0, 16)] = accs[16]
            plsc.store_scatter(fidx, [jnp.full((16,), nf, jnp.int32)],
                               jnp.full((16,), cur, jnp.int32), mask=lane0)

        def drain():
            pltpu.sync_copy(flq, accq.at[fidx], add=True)
            pltpu.sync_copy(flk, acck.at[fidx], add=True)
            pltpu.sync_copy(flc, accc.at[fidx], add=True)
            for k in range(FB // 16):
                fidx[pl.ds(k * 16, 16)] = psplat

        def process_chunk(p, carry):
            cur, nf, accs = carry[0], carry[1], carry[2:]
            bq, bt, ko = bufQ[p], batv[p], kkb[p]

            def grp_body(j, cr):
                cr = list(cr)
                bvec = bt[pl.ds(j * 16, 16)]
                for r in range(16):
                    cur_, nf_, ac = cr[0], cr[1], cr[2:]
                    i = j * 16 + r
                    b = bvec[r]
                    is_new = b != cur_

                    @pl.when(is_new)
                    def _():
                        flush_accs(nf_, cur_, ac)

                    qrows = [None] * 8
                    krows = [None] * 8
                    for k in range(4):
                        vq = plsc.bitcast(bq[i, pl.ds(k * 16, 16)], jnp.bfloat16)
                        qa, qb = plsc.unpack(vq, format=plsc.PackFormat.INTERLEAVED,
                                             preferred_element_type=jnp.float32)
                        qrows[k] = qa
                        qrows[k + 4] = qb
                        vk = plsc.bitcast(bq[i, pl.ds(64 + k * 16, 16)], jnp.bfloat16)
                        ka, kb = plsc.unpack(vk, format=plsc.PackFormat.INTERLEAVED,
                                             preferred_element_type=jnp.float32)
                        krows[k] = ka
                        krows[k + 4] = kb
                    for k in range(8):
                        ko[i, pl.ds(k * 16, 16)] = jnp.maximum(krows[k], 0.0)
                    rows = qrows + krows + [e0]
                    nac = [jnp.where(is_new, rw, a + rw) for rw, a in zip(rows, ac)]
                    cr = [b, nf_ + is_new.astype(jnp.int32)] + nac
                return tuple(cr)

            out = lax.fori_loop(0, CH // 16, grp_body, (cur, nf) + tuple(accs))
            cur, nf, accs = out[0], out[1], out[2:]

            @pl.when(nf >= 15)
            def _():
                drain()

            nf = jnp.where(nf >= 15, 0, nf)
            return (cur, nf) + tuple(accs)

        def half(p, n, carry):
            q = 1 - p
            @pl.when(n + 1 < nch)
            def _():
                pltpu.make_async_copy(map_hbm.at[pl.ds(0, CH)], idxv[q], isem[q]).wait()
                pltpu.make_async_copy(bat_hbm.at[pl.ds(0, CH)], batv[q], isem[q]).wait()

            @pl.when(n >= 1)
            def _():
                pltpu.make_async_copy(kkb[q], kk_hbm.at[pl.ds(0, CH)], wk[q]).wait()
                pltpu.make_async_copy(bufV[q], v_hbm.at[pl.ds(0, CH)], wv[q]).wait()

            @pl.when(n + 1 < nch)
            def _():
                pltpu.async_copy(hqk_hbm.at[idxv[q]], bufQ[q], gsem[q])
                pltpu.async_copy(hv_hbm.at[idxv[q]], bufV[q], gsem[q])

            pltpu.make_async_copy(hqk_hbm.at[idxv[p]], bufQ[p], gsem[p]).wait()
            pltpu.make_async_copy(hv_hbm.at[idxv[p]], bufV[p], gsem[p]).wait()

            carry = process_chunk(p, carry)

            base = wbase + n * CH
            pltpu.async_copy(kkb[p], kk_hbm.at[pl.ds(base, CH)], wk[p])
            pltpu.async_copy(bufV[p], v_hbm.at[pl.ds(base, CH)], wv[p])

            @pl.when(n + 2 < nch)
            def _():
                b2 = wbase + (n + 2) * CH
                pltpu.async_copy(map_hbm.at[pl.ds(b2, CH)], idxv[p], isem[p])
                pltpu.async_copy(bat_hbm.at[pl.ds(b2, CH)], batv[p], isem[p])

            return carry

        cur0 = bat0[pl.ds(0, 16)][0]
        init = (cur0, jnp.int32(0)) + tuple(jnp.zeros((16,), jnp.float32)
                                            for _ in range(17))

        def outer(g2, carry):
            carry = half(0, 2 * g2, carry)
            carry = half(1, 2 * g2 + 1, carry)
            return carry

        carry = lax.fori_loop(0, nch // 2, outer, init)
        cur, nf, accs = carry[0], carry[1], carry[2:]

        pltpu.make_async_copy(kkb1, kk_hbm.at[pl.ds(0, CH)], wk1).wait()
        pltpu.make_async_copy(bufV1, v_hbm.at[pl.ds(0, CH)], wv1).wait()

        flush_accs(nf, cur, list(accs))
        pltpu.sync_copy(flq, accq.at[fidx], add=True)
        pltpu.sync_copy(flk, acck.at[fidx], add=True)
        pltpu.sync_copy(flc, accc.at[fidx], add=True)
        plsc.subcore_barrier()

        for comp, acc in enumerate((accq, acck, accc)):
            pltpu.sync_copy(acc.at[pl.ds(s * 128, FB)], flq)
            pltpu.sync_copy(flq, parts_hbm.at[c, comp, pl.ds(s * 128, FB)])
            pltpu.sync_copy(acc.at[pl.ds(s * 128 + FB, 48)], flq.at[pl.ds(0, 48)])
            pltpu.sync_copy(flq.at[pl.ds(0, 48)],
                            parts_hbm.at[c, comp, pl.ds(s * 128 + FB, 48)])

    f = pl.kernel(
        body,
        out_type=(
            jax.ShapeDtypeStruct((S, D), jnp.float32),
            jax.ShapeDtypeStruct((S, D), jnp.float32),
            jax.ShapeDtypeStruct((NC, 3, AR, D), jnp.float32),
        ),
        mesh=mesh,
        compiler_params=pltpu.CompilerParams(needs_layout_passes=False),
        scratch_types=[
            pltpu.VMEM((CH, D), jnp.float32), pltpu.VMEM((CH, D), jnp.float32),
            pltpu.VMEM((CH, D), jnp.float32), pltpu.VMEM((CH, D), jnp.float32),
            pltpu.VMEM((CH, D), jnp.float32), pltpu.VMEM((CH, D), jnp.float32),
            pltpu.VMEM((CH,), jnp.int32), pltpu.VMEM((CH,), jnp.int32),
            pltpu.VMEM((CH,), jnp.int32), pltpu.VMEM((CH,), jnp.int32),
            pltpu.VMEM((FB, D), jnp.float32), pltpu.VMEM((FB, D), jnp.float32),
            pltpu.VMEM((FB, D), jnp.float32),
            pltpu.VMEM((FB,), jnp.int32),
            pltpu.VMEM_SHARED((AR, D), jnp.float32),
            pltpu.VMEM_SHARED((AR, D), jnp.float32),
            pltpu.VMEM_SHARED((AR, D), jnp.float32),
            pltpu.SemaphoreType.DMA, pltpu.SemaphoreType.DMA,
            pltpu.SemaphoreType.DMA, pltpu.SemaphoreType.DMA,
            pltpu.SemaphoreType.DMA, pltpu.SemaphoreType.DMA,
            pltpu.SemaphoreType.DMA, pltpu.SemaphoreType.DMA,
        ],
    )
    return f(hQK, hV, mapper, batch)


def _sc_edges(rows, cols, attr):
    mesh = plsc.VectorSubcoreMesh(core_axis_name="c", subcore_axis_name="s",
                                  num_cores=NC, num_subcores=NS)
    EPT = EC // NS
    EPW = EC // NW

    def body(row_hbm, col_hbm, attr_hbm, na_hbm,
             row4k, attr4k, hist, hist2, dinv, ident,
             row2k, col2k, attr2k, nbuf, degsp):
        c = lax.axis_index("c")
        s = lax.axis_index("s")
        w = c * NS + s

        iota = lax.iota(jnp.int32, 16)
        zero16 = jnp.zeros((16,), jnp.float32)

        def _z(i, _):
            hist[i, :] = zero16
            return 0
        lax.fori_loop(0, 128, _z, 0)

        ident[...] = iota

        @pl.when(s == 0)
        def _():
            def _z2(i, _):
                for k in range(8):
                    hist2[i, pl.ds(k * 16, 16)] = zero16
                return 0
            lax.fori_loop(0, 16, _z2, 0)
            pltpu.sync_copy(hist2, degsp)

        pltpu.sync_copy(row_hbm.at[pl.ds(s * EPT, EPT)], row4k)
        pltpu.sync_copy(attr_hbm.at[pl.ds(s * EPT, EPT)], attr4k)
        plsc.subcore_barrier()

        def _deg(j, _):
            rvec = row4k[pl.ds(j * 16, 16)]
            avec = attr4k[pl.ds(j * 16, 16)]
            for r in range(16):
                ri = rvec[r]
                hi = lax.shift_right_logical(ri, 4)
                lane = lax.bitwise_and(ri, 15)
                v = hist[hi, :]
                hist[hi, :] = v + jnp.where(iota == lane, avec[r], 0.0)
            return 0
        lax.fori_loop(0, EPT // 16, _deg, 0)

        def _rp(i, _):
            for k in range(8):
                hist2[i, pl.ds(k * 16, 16)] = hist[i * 8 + k, :]
            return 0
        lax.fori_loop(0, 16, _rp, 0)

        pltpu.sync_copy(hist2, degsp.at[ident], add=True)
        plsc.subcore_barrier()

        pltpu.sync_copy(degsp, hist2)
        half = jnp.full((16,), 0.5, jnp.float32)
        thalf = jnp.full((16,), 1.5, jnp.float32)
        magic = jnp.full((16,), 0x5F3759DF, jnp.int32)

        def _dv(i, _):
            for k in range(8):
                d = hist2[i, pl.ds(k * 16, 16)]
                y = plsc.bitcast(magic - lax.shift_right_logical(
                    plsc.bitcast(d, jnp.int32), 1), jnp.float32)
                hx = half * d
                for _ in range(3):
                    y = y * (thalf - hx * y * y)
                dinv[i, pl.ds(k * 16, 16)] = jnp.where(d > 0.0, y, 0.0)
            return 0
        lax.fori_loop(0, 16, _dv, 0)

        base = w * EPW
        pltpu.sync_copy(row_hbm.at[pl.ds(base, EPW)], row2k)
        pltpu.sync_copy(col_hbm.at[pl.ds(base, EPW)], col2k)
        pltpu.sync_copy(attr_hbm.at[pl.ds(base, EPW)], attr2k)

        def _na(i, _):
            r16 = row2k[pl.ds(i * 16, 16)]
            c16 = col2k[pl.ds(i * 16, 16)]
            a16 = attr2k[pl.ds(i * 16, 16)]
            dr = plsc.load_gather(dinv, [lax.shift_right_logical(r16, 7),
                                         lax.bitwise_and(r16, 127)])
            dc = plsc.load_gather(dinv, [lax.shift_right_logical(c16, 7),
                                         lax.bitwise_and(c16, 127)])
            nbuf[pl.ds(i * 16, 16)] = dr * a16 * dc
            return 0
        lax.fori_loop(0, EPW // 16, _na, 0)

        pltpu.sync_copy(nbuf, na_hbm.at[pl.ds(base, EPW)])

    f = pl.kernel(
        body,
        out_type=jax.ShapeDtypeStruct((EC,), jnp.float32),
        mesh=mesh,
        compiler_params=pltpu.CompilerParams(needs_layout_passes=False),
        scratch_types=[
            pltpu.VMEM((EPT,), jnp.int32), pltpu.VMEM((EPT,), jnp.float32),
            pltpu.VMEM((128, 16), jnp.float32), pltpu.VMEM((16, 128), jnp.float32),
            pltpu.VMEM((16, 128), jnp.float32),
            pltpu.VMEM((16,), jnp.int32),
            pltpu.VMEM((EPW,), jnp.int32), pltpu.VMEM((EPW,), jnp.int32),
            pltpu.VMEM((EPW,), jnp.float32), pltpu.VMEM((EPW,), jnp.float32),
            pltpu.VMEM_SHARED((16, 128), jnp.float32),
        ],
    )
    return f(rows, cols, attr)


def _dense_nodes(x, Wt1, bt1, Wt2, bt2, Wpre1, bpre1, Wpre2, bpre2, Wk, bk, Wv, bv):
    BLK = 1000
    grid = N // BLK

    def _pack2(t):
        lo = lax.bitcast_convert_type(t[:, :64].astype(jnp.bfloat16),
                                      jnp.uint16).astype(jnp.uint32)
        hi = lax.bitcast_convert_type(t[:, 64:].astype(jnp.bfloat16),
                                      jnp.uint16).astype(jnp.uint32)
        return lax.bitcast_convert_type(lo | (hi << 16), jnp.float32)

    def _bdot(a, b):
        return jnp.dot(a.astype(jnp.bfloat16), b.astype(jnp.bfloat16),
                       preferred_element_type=jnp.float32)

    def body(x_ref, wt1, bt1_, wt2, bt2_, wp1, bp1_, wp2, bp2_, wk, bk_, wv, bv_,
             hqk, hv):
        xb = x_ref[...]
        h = jnp.maximum(_bdot(xb, wt1[...]) + bt1_[...], 0.0)
        h = jnp.maximum(_bdot(h, wt2[...]) + bt2_[...], 0.0)
        q = jnp.maximum(_bdot(h, wp1[...]) + bp1_[...], 0.0)
        pq = _pack2(_bdot(q, wp2[...]) + bp2_[...])
        pk = _pack2(_bdot(h, wk[...]) + bk_[...])
        hqk[...] = jnp.concatenate([pq, pk], axis=1)
        hv[...] = _bdot(h, wv[...]) + bv_[...]

    wspec = pl.BlockSpec((D, D), lambda i: (0, 0))
    bspec = pl.BlockSpec((1, D), lambda i: (0, 0))
    xspec = pl.BlockSpec((BLK, D), lambda i: (i, 0))
    return pl.pallas_call(
        body,
        grid=(grid,),
        in_specs=[xspec] + [wspec, bspec] * 6,
        out_specs=[xspec, xspec],
        out_shape=[jax.ShapeDtypeStruct((N, D), jnp.float32)] * 2,
    )(x, Wt1, bt1.reshape(1, D), Wt2, bt2.reshape(1, D),
      Wpre1, bpre1.reshape(1, D), Wpre2, bpre2.reshape(1, D),
      Wk, bk.reshape(1, D), Wv, bv.reshape(1, D))


def _post(parts, Wpost1, bpost1, Wpost2, bpost2):
    def body(pr, w1, b1, w2, b2, kq, ksk):
        comb = pr[0] + pr[1]
        sq = comb[0, :P]
        sks = comb[1, :P]
        cnt = comb[2, :P, 0:1]
        t = jnp.maximum(jnp.dot(sq, w1[...], preferred_element_type=jnp.float32) + b1[...], 0.0)
        t = jnp.dot(t, w2[...], preferred_element_type=jnp.float32) + b2[...]
        kq[...] = jnp.maximum(t, 0.0)
        ksk[...] = jnp.maximum(sks / jnp.maximum(cnt, 1.0), 0.0)

    return pl.pallas_call(
        body,
        grid=(1,),
        in_specs=[
            pl.BlockSpec((NC, 3, AR, D), lambda i: (0, 0, 0, 0)),
            pl.BlockSpec((D, D), lambda i: (0, 0)),
            pl.BlockSpec((1, D), lambda i: (0, 0)),
            pl.BlockSpec((D, D), lambda i: (0, 0)),
            pl.BlockSpec((1, D), lambda i: (0, 0)),
        ],
        out_specs=[pl.BlockSpec((P, D), lambda i: (0, 0))] * 2,
        out_shape=[jax.ShapeDtypeStruct((P, D), jnp.float32)] * 2,
    )(parts, Wpost1, bpost1.reshape(1, D), Wpost2, bpost2.reshape(1, D))


def kernel(x, subgraphs_nodes_mapper, subgraphs_batch, subgraphs_batch_row,
           subgraphs_batch_col, coarsen_edge_attr, Wt1, bt1, Wt2, bt2,
           Wpre1, bpre1, Wpre2, bpre2, Wpost1, bpost1, Wpost2, bpost2,
           Wk, bk, Wv, bv):
    mapper = subgraphs_nodes_mapper.astype(jnp.int32)
    batch = subgraphs_batch.astype(jnp.int32)
    erow = subgraphs_batch_row.astype(jnp.int32)
    ecol = subgraphs_batch_col.astype(jnp.int32)

    hQK, hV = _dense_nodes(x, Wt1, bt1, Wt2, bt2, Wpre1, bpre1, Wpre2, bpre2,
                           Wk, bk, Wv, bv)
    norm_attr = _sc_edges(erow, ecol, coarsen_edge_attr)
    kK, V, parts = _sc_main(hQK, hV, mapper, batch)
    kQ, ksK = _post(parts, Wpost1, bpost1, Wpost2, bpost2)
    return (kQ, kK, ksK, V, norm_attr)


# speedup vs baseline: 1.0340x; 1.0340x over previous
import jax
import jax.numpy as jnp
from jax import lax
from jax.experimental import pallas as pl
from jax.experimental.pallas import tpu as pltpu
from jax.experimental.pallas import tpu_sc as plsc

N = 100000
D = 128
S = 400000
P = 2000
EC = 64000

NC = 2
NS = 16
NW = NC * NS

CH = 64
WPW = 12544
NCH_FULL = WPW // CH
NCH_LAST = (S - (NW - 1) * WPW) // CH
AR = 2048
FB = 80


def _sc_main(hQK, hV, mapper, batch):
    mesh = plsc.VectorSubcoreMesh(core_axis_name="c", subcore_axis_name="s",
                                  num_cores=NC, num_subcores=NS)

    def body(hqk_hbm, hv_hbm, map_hbm, bat_hbm,
             kk_hbm, v_hbm, parts_hbm,
             bufQ0, bufQ1, bufV0, bufV1, kkb0, kkb1,
             idx0, idx1, bat0, bat1, flq, flk, flc, fidx,
             accq, acck, accc,
             gsem0, gsem1, isem0, isem1, wk0, wk1, wv0, wv1):
        c = lax.axis_index("c")
        s = lax.axis_index("s")
        w = c * NS + s
        wbase = w * WPW
        nch = jnp.where(w == NW - 1, NCH_LAST, NCH_FULL)

        bufQ = (bufQ0, bufQ1)
        bufV = (bufV0, bufV1)
        kkb = (kkb0, kkb1)
        idxv = (idx0, idx1)
        batv = (bat0, bat1)
        gsem = (gsem0, gsem1)
        isem = (isem0, isem1)
        wk = (wk0, wk1)
        wv = (wv0, wv1)

        pltpu.sync_copy(map_hbm.at[pl.ds(wbase, CH)], idx0)
        pltpu.sync_copy(bat_hbm.at[pl.ds(wbase, CH)], bat0)
        pltpu.async_copy(hqk_hbm.at[idx0], bufQ0, gsem0)
        pltpu.async_copy(hv_hbm.at[idx0], bufV0, gsem0)
        pltpu.async_copy(map_hbm.at[pl.ds(wbase + CH, CH)], idx1, isem1)
        pltpu.async_copy(bat_hbm.at[pl.ds(wbase + CH, CH)], bat1, isem1)

        zero16 = jnp.zeros((16,), jnp.float32)

        def _zf(i, _):
            for k in range(8):
                flq[i, pl.ds(k * 16, 16)] = zero16
            return 0
        lax.fori_loop(0, FB, _zf, 0)
        for acc in (accq, acck, accc):
            pltpu.sync_copy(flq, acc.at[pl.ds(s * 128, FB)])
            pltpu.sync_copy(flq.at[pl.ds(0, 48)], acc.at[pl.ds(s * 128 + FB, 48)])
        psplat = jnp.full((16,), P, jnp.int32)
        for k in range(FB // 16):
            fidx[pl.ds(k * 16, 16)] = psplat
        plsc.subcore_barrier()

        iota16 = lax.iota(jnp.int32, 16)
        lane0 = iota16 == 0
        e0 = jnp.where(lane0, 1.0, 0.0).astype(jnp.float32)

        def flush_accs(nf, cur, accs):
            for k in range(8):
                flq[nf, pl.ds(k * 16, 16)] = accs[k]
            for k in range(8):
                flk[nf, pl.ds(k * 16, 16)] = accs[8 + k]
            flc[nf, pl.ds(0, 16)] = accs[16]
            plsc.store_scatter(fidx, [jnp.full((16,), nf, jnp.int32)],
                               jnp.full((16,), cur, jnp.int32), mask=lane0)

        def drain():
            pltpu.sync_copy(flq, accq.at[fidx], add=True)
            pltpu.sync_copy(flk, acck.at[fidx], add=True)
            pltpu.sync_copy(flc, accc.at[fidx], add=True)
            for k in range(FB // 16):
                fidx[pl.ds(k * 16, 16)] = psplat

        def process_chunk(p, carry):
            cur, nf, accs = carry[0], carry[1], carry[2:]
            bq, bt, ko = bufQ[p], batv[p], kkb[p]

            def grp_body(j, cr):
                cr = list(cr)
                bvec = bt[pl.ds(j * 16, 16)]
                for r in range(16):
                    cur_, nf_, ac = cr[0], cr[1], cr[2:]
                    i = j * 16 + r
                    b = bvec[r]
                    is_new = b != cur_

                    @pl.when(is_new)
                    def _():
                        flush_accs(nf_, cur_, ac)

                    qrows = [None] * 8
                    krows = [None] * 8
                    for k in range(4):
                        vq = plsc.bitcast(bq[i, pl.ds(k * 16, 16)], jnp.bfloat16)
                        qa, qb = plsc.unpack(vq, format=plsc.PackFormat.INTERLEAVED,
                                             preferred_element_type=jnp.float32)
                        qrows[k] = qa
                        qrows[k + 4] = qb
                        vk = plsc.bitcast(bq[i, pl.ds(64 + k * 16, 16)], jnp.bfloat16)
                        ka, kb = plsc.unpack(vk, format=plsc.PackFormat.INTERLEAVED,
                                             preferred_element_type=jnp.float32)
                        krows[k] = ka
                        krows[k + 4] = kb
                    for k in range(8):
                        ko[i, pl.ds(k * 16, 16)] = jnp.maximum(krows[k], 0.0)
                    rows = qrows + krows + [e0]
                    nac = [jnp.where(is_new, rw, a + rw) for rw, a in zip(rows, ac)]
                    cr = [b, nf_ + is_new.astype(jnp.int32)] + nac
                return tuple(cr)

            out = lax.fori_loop(0, CH // 16, grp_body, (cur, nf) + tuple(accs))
            cur, nf, accs = out[0], out[1], out[2:]

            @pl.when(nf >= 15)
            def _():
                drain()

            nf = jnp.where(nf >= 15, 0, nf)
            return (cur, nf) + tuple(accs)

        def half(p, n, carry):
            q = 1 - p
            @pl.when(n + 1 < nch)
            def _():
                pltpu.make_async_copy(map_hbm.at[pl.ds(0, CH)], idxv[q], isem[q]).wait()
                pltpu.make_async_copy(bat_hbm.at[pl.ds(0, CH)], batv[q], isem[q]).wait()

            @pl.when(n >= 1)
            def _():
                pltpu.make_async_copy(kkb[q], kk_hbm.at[pl.ds(0, CH)], wk[q]).wait()
                pltpu.make_async_copy(bufV[q], v_hbm.at[pl.ds(0, CH)], wv[q]).wait()

            @pl.when(n + 1 < nch)
            def _():
                pltpu.async_copy(hqk_hbm.at[idxv[q]], bufQ[q], gsem[q])
                pltpu.async_copy(hv_hbm.at[idxv[q]], bufV[q], gsem[q])

            pltpu.make_async_copy(hqk_hbm.at[idxv[p]], bufQ[p], gsem[p]).wait()
            pltpu.make_async_copy(hv_hbm.at[idxv[p]], bufV[p], gsem[p]).wait()

            carry = process_chunk(p, carry)

            base = wbase + n * CH
            pltpu.async_copy(kkb[p], kk_hbm.at[pl.ds(base, CH)], wk[p])
            pltpu.async_copy(bufV[p], v_hbm.at[pl.ds(base, CH)], wv[p])

            @pl.when(n + 2 < nch)
            def _():
                b2 = wbase + (n + 2) * CH
                pltpu.async_copy(map_hbm.at[pl.ds(b2, CH)], idxv[p], isem[p])
                pltpu.async_copy(bat_hbm.at[pl.ds(b2, CH)], batv[p], isem[p])

            return carry

        cur0 = bat0[pl.ds(0, 16)][0]
        init = (cur0, jnp.int32(0)) + tuple(jnp.zeros((16,), jnp.float32)
                                            for _ in range(17))

        def outer(g2, carry):
            carry = half(0, 2 * g2, carry)
            carry = half(1, 2 * g2 + 1, carry)
            return carry

        carry = lax.fori_loop(0, nch // 2, outer, init)
        cur, nf, accs = carry[0], carry[1], carry[2:]

        pltpu.make_async_copy(kkb1, kk_hbm.at[pl.ds(0, CH)], wk1).wait()
        pltpu.make_async_copy(bufV1, v_hbm.at[pl.ds(0, CH)], wv1).wait()

        flush_accs(nf, cur, list(accs))
        pltpu.sync_copy(flq, accq.at[fidx], add=True)
        pltpu.sync_copy(flk, acck.at[fidx], add=True)
        pltpu.sync_copy(flc, accc.at[fidx], add=True)
        plsc.subcore_barrier()

        for comp, acc in enumerate((accq, acck, accc)):
            pltpu.sync_copy(acc.at[pl.ds(s * 128, FB)], flq)
            pltpu.sync_copy(flq, parts_hbm.at[c, comp, pl.ds(s * 128, FB)])
            pltpu.sync_copy(acc.at[pl.ds(s * 128 + FB, 48)], flq.at[pl.ds(0, 48)])
            pltpu.sync_copy(flq.at[pl.ds(0, 48)],
                            parts_hbm.at[c, comp, pl.ds(s * 128 + FB, 48)])

    f = pl.kernel(
        body,
        out_type=(
            jax.ShapeDtypeStruct((S, D), jnp.float32),
            jax.ShapeDtypeStruct((S, D), jnp.float32),
            jax.ShapeDtypeStruct((NC, 3, AR, D), jnp.float32),
        ),
        mesh=mesh,
        compiler_params=pltpu.CompilerParams(needs_layout_passes=False),
        scratch_types=[
            pltpu.VMEM((CH, D), jnp.float32), pltpu.VMEM((CH, D), jnp.float32),
            pltpu.VMEM((CH, D), jnp.float32), pltpu.VMEM((CH, D), jnp.float32),
            pltpu.VMEM((CH, D), jnp.float32), pltpu.VMEM((CH, D), jnp.float32),
            pltpu.VMEM((CH,), jnp.int32), pltpu.VMEM((CH,), jnp.int32),
            pltpu.VMEM((CH,), jnp.int32), pltpu.VMEM((CH,), jnp.int32),
            pltpu.VMEM((FB, D), jnp.float32), pltpu.VMEM((FB, D), jnp.float32),
            pltpu.VMEM((FB, D), jnp.float32),
            pltpu.VMEM((FB,), jnp.int32),
            pltpu.VMEM_SHARED((AR, D), jnp.float32),
            pltpu.VMEM_SHARED((AR, D), jnp.float32),
            pltpu.VMEM_SHARED((AR, D), jnp.float32),
            pltpu.SemaphoreType.DMA, pltpu.SemaphoreType.DMA,
            pltpu.SemaphoreType.DMA, pltpu.SemaphoreType.DMA,
            pltpu.SemaphoreType.DMA, pltpu.SemaphoreType.DMA,
            pltpu.SemaphoreType.DMA, pltpu.SemaphoreType.DMA,
        ],
    )
    return f(hQK, hV, mapper, batch)


def _sc_edges(rows, cols, attr):
    mesh = plsc.VectorSubcoreMesh(core_axis_name="c", subcore_axis_name="s",
                                  num_cores=NC, num_subcores=NS)
    EPT = EC // NS
    EPW = EC // NW

    def body(row_hbm, col_hbm, attr_hbm, na_hbm,
             row4k, attr4k, hist, hist2, dinv, ident,
             row2k, col2k, attr2k, nbuf, degsp):
        c = lax.axis_index("c")
        s = lax.axis_index("s")
        w = c * NS + s

        iota = lax.iota(jnp.int32, 16)
        zero16 = jnp.zeros((16,), jnp.float32)

        def _z(i, _):
            hist[i, :] = zero16
            return 0
        lax.fori_loop(0, 128, _z, 0)

        ident[...] = iota

        @pl.when(s == 0)
        def _():
            def _z2(i, _):
                for k in range(8):
                    hist2[i, pl.ds(k * 16, 16)] = zero16
                return 0
            lax.fori_loop(0, 16, _z2, 0)
            pltpu.sync_copy(hist2, degsp)

        pltpu.sync_copy(row_hbm.at[pl.ds(s * EPT, EPT)], row4k)
        pltpu.sync_copy(attr_hbm.at[pl.ds(s * EPT, EPT)], attr4k)
        plsc.subcore_barrier()

        def _deg(j, _):
            rvec = row4k[pl.ds(j * 16, 16)]
            avec = attr4k[pl.ds(j * 16, 16)]
            plsc.addupdate_scatter(
                hist, [lax.shift_right_logical(rvec, 4),
                       lax.bitwise_and(rvec, 15)], avec)
            return 0
        lax.fori_loop(0, EPT // 16, _deg, 0)

        def _rp(i, _):
            for k in range(8):
                hist2[i, pl.ds(k * 16, 16)] = hist[i * 8 + k, :]
            return 0
        lax.fori_loop(0, 16, _rp, 0)

        pltpu.sync_copy(hist2, degsp.at[ident], add=True)
        plsc.subcore_barrier()

        pltpu.sync_copy(degsp, hist2)
        half = jnp.full((16,), 0.5, jnp.float32)
        thalf = jnp.full((16,), 1.5, jnp.float32)
        magic = jnp.full((16,), 0x5F3759DF, jnp.int32)

        def _dv(i, _):
            for k in range(8):
                d = hist2[i, pl.ds(k * 16, 16)]
                y = plsc.bitcast(magic - lax.shift_right_logical(
                    plsc.bitcast(d, jnp.int32), 1), jnp.float32)
                hx = half * d
                for _ in range(3):
                    y = y * (thalf - hx * y * y)
                dinv[i, pl.ds(k * 16, 16)] = jnp.where(d > 0.0, y, 0.0)
            return 0
        lax.fori_loop(0, 16, _dv, 0)

        base = w * EPW
        pltpu.sync_copy(row_hbm.at[pl.ds(base, EPW)], row2k)
        pltpu.sync_copy(col_hbm.at[pl.ds(base, EPW)], col2k)
        pltpu.sync_copy(attr_hbm.at[pl.ds(base, EPW)], attr2k)

        def _na(i, _):
            r16 = row2k[pl.ds(i * 16, 16)]
            c16 = col2k[pl.ds(i * 16, 16)]
            a16 = attr2k[pl.ds(i * 16, 16)]
            dr = plsc.load_gather(dinv, [lax.shift_right_logical(r16, 7),
                                         lax.bitwise_and(r16, 127)])
            dc = plsc.load_gather(dinv, [lax.shift_right_logical(c16, 7),
                                         lax.bitwise_and(c16, 127)])
            nbuf[pl.ds(i * 16, 16)] = dr * a16 * dc
            return 0
        lax.fori_loop(0, EPW // 16, _na, 0)

        pltpu.sync_copy(nbuf, na_hbm.at[pl.ds(base, EPW)])

    f = pl.kernel(
        body,
        out_type=jax.ShapeDtypeStruct((EC,), jnp.float32),
        mesh=mesh,
        compiler_params=pltpu.CompilerParams(needs_layout_passes=False),
        scratch_types=[
            pltpu.VMEM((EPT,), jnp.int32), pltpu.VMEM((EPT,), jnp.float32),
            pltpu.VMEM((128, 16), jnp.float32), pltpu.VMEM((16, 128), jnp.float32),
            pltpu.VMEM((16, 128), jnp.float32),
            pltpu.VMEM((16,), jnp.int32),
            pltpu.VMEM((EPW,), jnp.int32), pltpu.VMEM((EPW,), jnp.int32),
            pltpu.VMEM((EPW,), jnp.float32), pltpu.VMEM((EPW,), jnp.float32),
            pltpu.VMEM_SHARED((16, 128), jnp.float32),
        ],
    )
    return f(rows, cols, attr)


def _dense_nodes(x, Wt1, bt1, Wt2, bt2, Wpre1, bpre1, Wpre2, bpre2, Wk, bk, Wv, bv):
    BLK = 1000
    grid = N // BLK

    def _pack2(t):
        lo = lax.bitcast_convert_type(t[:, :64].astype(jnp.bfloat16),
                                      jnp.uint16).astype(jnp.uint32)
        hi = lax.bitcast_convert_type(t[:, 64:].astype(jnp.bfloat16),
                                      jnp.uint16).astype(jnp.uint32)
        return lax.bitcast_convert_type(lo | (hi << 16), jnp.float32)

    def _bdot(a, b):
        return jnp.dot(a.astype(jnp.bfloat16), b.astype(jnp.bfloat16),
                       preferred_element_type=jnp.float32)

    def body(x_ref, wt1, bt1_, wt2, bt2_, wp1, bp1_, wp2, bp2_, wk, bk_, wv, bv_,
             hqk, hv):
        xb = x_ref[...]
        h = jnp.maximum(_bdot(xb, wt1[...]) + bt1_[...], 0.0)
        h = jnp.maximum(_bdot(h, wt2[...]) + bt2_[...], 0.0)
        q = jnp.maximum(_bdot(h, wp1[...]) + bp1_[...], 0.0)
        pq = _pack2(_bdot(q, wp2[...]) + bp2_[...])
        pk = _pack2(_bdot(h, wk[...]) + bk_[...])
        hqk[...] = jnp.concatenate([pq, pk], axis=1)
        hv[...] = _bdot(h, wv[...]) + bv_[...]

    wspec = pl.BlockSpec((D, D), lambda i: (0, 0))
    bspec = pl.BlockSpec((1, D), lambda i: (0, 0))
    xspec = pl.BlockSpec((BLK, D), lambda i: (i, 0))
    return pl.pallas_call(
        body,
        grid=(grid,),
        in_specs=[xspec] + [wspec, bspec] * 6,
        out_specs=[xspec, xspec],
        out_shape=[jax.ShapeDtypeStruct((N, D), jnp.float32)] * 2,
    )(x, Wt1, bt1.reshape(1, D), Wt2, bt2.reshape(1, D),
      Wpre1, bpre1.reshape(1, D), Wpre2, bpre2.reshape(1, D),
      Wk, bk.reshape(1, D), Wv, bv.reshape(1, D))


def _post(parts, Wpost1, bpost1, Wpost2, bpost2):
    def body(pr, w1, b1, w2, b2, kq, ksk):
        comb = pr[0] + pr[1]
        sq = comb[0, :P]
        sks = comb[1, :P]
        cnt = comb[2, :P, 0:1]
        t = jnp.maximum(jnp.dot(sq, w1[...], preferred_element_type=jnp.float32) + b1[...], 0.0)
        t = jnp.dot(t, w2[...], preferred_element_type=jnp.float32) + b2[...]
        kq[...] = jnp.maximum(t, 0.0)
        ksk[...] = jnp.maximum(sks / jnp.maximum(cnt, 1.0), 0.0)

    return pl.pallas_call(
        body,
        grid=(1,),
        in_specs=[
            pl.BlockSpec((NC, 3, AR, D), lambda i: (0, 0, 0, 0)),
            pl.BlockSpec((D, D), lambda i: (0, 0)),
            pl.BlockSpec((1, D), lambda i: (0, 0)),
            pl.BlockSpec((D, D), lambda i: (0, 0)),
            pl.BlockSpec((1, D), lambda i: (0, 0)),
        ],
        out_specs=[pl.BlockSpec((P, D), lambda i: (0, 0))] * 2,
        out_shape=[jax.ShapeDtypeStruct((P, D), jnp.float32)] * 2,
    )(parts, Wpost1, bpost1.reshape(1, D), Wpost2, bpost2.reshape(1, D))


def kernel(x, subgraphs_nodes_mapper, subgraphs_batch, subgraphs_batch_row,
           subgraphs_batch_col, coarsen_edge_attr, Wt1, bt1, Wt2, bt2,
           Wpre1, bpre1, Wpre2, bpre2, Wpost1, bpost1, Wpost2, bpost2,
           Wk, bk, Wv, bv):
    mapper = subgraphs_nodes_mapper.astype(jnp.int32)
    batch = subgraphs_batch.astype(jnp.int32)
    erow = subgraphs_batch_row.astype(jnp.int32)
    ecol = subgraphs_batch_col.astype(jnp.int32)

    hQK, hV = _dense_nodes(x, Wt1, bt1, Wt2, bt2, Wpre1, bpre1, Wpre2, bpre2,
                           Wk, bk, Wv, bv)
    norm_attr = _sc_edges(erow, ecol, coarsen_edge_attr)
    kK, V, parts = _sc_main(hQK, hV, mapper, batch)
    kQ, ksK = _post(parts, Wpost1, bpost1, Wpost2, bpost2)
    return (kQ, kK, ksK, V, norm_attr)


# speedup vs baseline: 1.1014x; 1.0651x over previous
import jax
import jax.numpy as jnp
from jax import lax
from jax.experimental import pallas as pl
from jax.experimental.pallas import tpu as pltpu
from jax.experimental.pallas import tpu_sc as plsc

N = 100000
D = 128
S = 400000
P = 2000
EC = 64000

NC = 2
NS = 16
NW = NC * NS

CH = 64
WPW = 12544
NCH_FULL = WPW // CH
NCH_LAST = (S - (NW - 1) * WPW) // CH
AR = 2048
FB = 80


def _sc_main(hQK, hV, mapper, batch):
    mesh = plsc.VectorSubcoreMesh(core_axis_name="c", subcore_axis_name="s",
                                  num_cores=NC, num_subcores=NS)

    def body(hqk_hbm, hv_hbm, map_hbm, bat_hbm,
             kk_hbm, v_hbm, parts_hbm,
             bufQ0, bufQ1, bufV0, bufV1, kkb0, kkb1,
             idx0, idx1, bat0, bat1, flq, flk, flc, fidx,
             accq, acck, accc,
             gsem0, gsem1, isem0, isem1, wk0, wk1, wv0, wv1):
        c = lax.axis_index("c")
        s = lax.axis_index("s")
        w = c * NS + s
        wbase = w * WPW
        nch = jnp.where(w == NW - 1, NCH_LAST, NCH_FULL)

        bufQ = (bufQ0, bufQ1)
        bufV = (bufV0, bufV1)
        kkb = (kkb0, kkb1)
        idxv = (idx0, idx1)
        batv = (bat0, bat1)
        gsem = (gsem0, gsem1)
        isem = (isem0, isem1)
        wk = (wk0, wk1)
        wv = (wv0, wv1)

        pltpu.sync_copy(map_hbm.at[pl.ds(wbase, CH)], idx0)
        pltpu.sync_copy(bat_hbm.at[pl.ds(wbase, CH)], bat0)
        pltpu.async_copy(hqk_hbm.at[idx0], bufQ0, gsem0)
        pltpu.async_copy(hv_hbm.at[idx0], bufV0, gsem0)
        pltpu.async_copy(map_hbm.at[pl.ds(wbase + CH, CH)], idx1, isem1)
        pltpu.async_copy(bat_hbm.at[pl.ds(wbase + CH, CH)], bat1, isem1)

        zero16 = jnp.zeros((16,), jnp.float32)

        def _zf(i, _):
            for k in range(8):
                flq[i, pl.ds(k * 16, 16)] = zero16
            return 0
        lax.fori_loop(0, FB, _zf, 0)
        for acc in (accq, acck, accc):
            pltpu.sync_copy(flq, acc.at[pl.ds(s * 128, FB)])
            pltpu.sync_copy(flq.at[pl.ds(0, 48)], acc.at[pl.ds(s * 128 + FB, 48)])
        psplat = jnp.full((16,), P, jnp.int32)
        for k in range(FB // 16):
            fidx[pl.ds(k * 16, 16)] = psplat
        plsc.subcore_barrier()

        iota16 = lax.iota(jnp.int32, 16)
        lane0 = iota16 == 0
        e0 = jnp.where(lane0, 1.0, 0.0).astype(jnp.float32)

        def flush_accs(nf, cur, accs):
            for k in range(8):
                flq[nf, pl.ds(k * 16, 16)] = accs[k]
            for k in range(8):
                flk[nf, pl.ds(k * 16, 16)] = accs[8 + k]
            flc[nf, pl.ds(0, 16)] = accs[16]
            plsc.store_scatter(fidx, [jnp.full((16,), nf, jnp.int32)],
                               jnp.full((16,), cur, jnp.int32), mask=lane0)

        def drain():
            pltpu.sync_copy(flq, accq.at[fidx], add=True)
            pltpu.sync_copy(flk, acck.at[fidx], add=True)
            pltpu.sync_copy(flc, accc.at[fidx], add=True)
            for k in range(FB // 16):
                fidx[pl.ds(k * 16, 16)] = psplat

        def process_chunk(p, carry):
            cur, nf, accs = carry[0], carry[1], carry[2:]
            bq, bt, ko = bufQ[p], batv[p], kkb[p]

            def grp_body(j, cr):
                cr = list(cr)
                bvec = bt[pl.ds(j * 16, 16)]
                for r in range(16):
                    cur_, nf_, ac = cr[0], cr[1], cr[2:]
                    i = j * 16 + r
                    b = bvec[r]
                    is_new = b != cur_

                    @pl.when(is_new)
                    def _():
                        flush_accs(nf_, cur_, ac)

                    qrows = [None] * 8
                    krows = [None] * 8
                    for k in range(4):
                        vq = plsc.bitcast(bq[i, pl.ds(k * 16, 16)], jnp.bfloat16)
                        qa, qb = plsc.unpack(vq, format=plsc.PackFormat.INTERLEAVED,
                                             preferred_element_type=jnp.float32)
                        qrows[k] = qa
                        qrows[k + 4] = qb
                        vk = plsc.bitcast(bq[i, pl.ds(64 + k * 16, 16)], jnp.bfloat16)
                        ka, kb = plsc.unpack(vk, format=plsc.PackFormat.INTERLEAVED,
                                             preferred_element_type=jnp.float32)
                        krows[k] = ka
                        krows[k + 4] = kb
                    for k in range(8):
                        ko[i, pl.ds(k * 16, 16)] = jnp.maximum(krows[k], 0.0)
                    rows = qrows + krows + [e0]
                    nac = [jnp.where(is_new, rw, a + rw) for rw, a in zip(rows, ac)]
                    cr = [b, nf_ + is_new.astype(jnp.int32)] + nac
                return tuple(cr)

            out = lax.fori_loop(0, CH // 16, grp_body, (cur, nf) + tuple(accs))
            cur, nf, accs = out[0], out[1], out[2:]

            @pl.when(nf >= 15)
            def _():
                drain()

            nf = jnp.where(nf >= 15, 0, nf)
            return (cur, nf) + tuple(accs)

        def half(p, n, carry):
            q = 1 - p
            @pl.when(n + 1 < nch)
            def _():
                pltpu.make_async_copy(map_hbm.at[pl.ds(0, CH)], idxv[q], isem[q]).wait()
                pltpu.make_async_copy(bat_hbm.at[pl.ds(0, CH)], batv[q], isem[q]).wait()

            @pl.when(n >= 1)
            def _():
                pltpu.make_async_copy(kkb[q], kk_hbm.at[pl.ds(0, CH)], wk[q]).wait()
                pltpu.make_async_copy(bufV[q], v_hbm.at[pl.ds(0, CH)], wv[q]).wait()

            @pl.when(n + 1 < nch)
            def _():
                pltpu.async_copy(hqk_hbm.at[idxv[q]], bufQ[q], gsem[q])
                pltpu.async_copy(hv_hbm.at[idxv[q]], bufV[q], gsem[q])

            pltpu.make_async_copy(hqk_hbm.at[idxv[p]], bufQ[p], gsem[p]).wait()
            pltpu.make_async_copy(hv_hbm.at[idxv[p]], bufV[p], gsem[p]).wait()

            carry = process_chunk(p, carry)

            base = wbase + n * CH
            pltpu.async_copy(kkb[p], kk_hbm.at[pl.ds(base, CH)], wk[p])
            pltpu.async_copy(bufV[p], v_hbm.at[pl.ds(base, CH)], wv[p])

            @pl.when(n + 2 < nch)
            def _():
                b2 = wbase + (n + 2) * CH
                pltpu.async_copy(map_hbm.at[pl.ds(b2, CH)], idxv[p], isem[p])
                pltpu.async_copy(bat_hbm.at[pl.ds(b2, CH)], batv[p], isem[p])

            return carry

        cur0 = bat0[pl.ds(0, 16)][0]
        init = (cur0, jnp.int32(0)) + tuple(jnp.zeros((16,), jnp.float32)
                                            for _ in range(17))

        def outer(g2, carry):
            carry = half(0, 2 * g2, carry)
            carry = half(1, 2 * g2 + 1, carry)
            return carry

        carry = lax.fori_loop(0, nch // 2, outer, init)
        cur, nf, accs = carry[0], carry[1], carry[2:]

        pltpu.make_async_copy(kkb1, kk_hbm.at[pl.ds(0, CH)], wk1).wait()
        pltpu.make_async_copy(bufV1, v_hbm.at[pl.ds(0, CH)], wv1).wait()

        flush_accs(nf, cur, list(accs))
        pltpu.sync_copy(flq, accq.at[fidx], add=True)
        pltpu.sync_copy(flk, acck.at[fidx], add=True)
        pltpu.sync_copy(flc, accc.at[fidx], add=True)
        plsc.subcore_barrier()

        for comp, acc in enumerate((accq, acck, accc)):
            pltpu.sync_copy(acc.at[pl.ds(s * 128, FB)], flq)
            pltpu.sync_copy(flq, parts_hbm.at[c, comp, pl.ds(s * 128, FB)])
            pltpu.sync_copy(acc.at[pl.ds(s * 128 + FB, 48)], flq.at[pl.ds(0, 48)])
            pltpu.sync_copy(flq.at[pl.ds(0, 48)],
                            parts_hbm.at[c, comp, pl.ds(s * 128 + FB, 48)])

    f = pl.kernel(
        body,
        out_type=(
            jax.ShapeDtypeStruct((S, D), jnp.float32),
            jax.ShapeDtypeStruct((S, D), jnp.float32),
            jax.ShapeDtypeStruct((NC, 3, AR, D), jnp.float32),
        ),
        mesh=mesh,
        compiler_params=pltpu.CompilerParams(needs_layout_passes=False),
        scratch_types=[
            pltpu.VMEM((CH, D), jnp.float32), pltpu.VMEM((CH, D), jnp.float32),
            pltpu.VMEM((CH, D), jnp.float32), pltpu.VMEM((CH, D), jnp.float32),
            pltpu.VMEM((CH, D), jnp.float32), pltpu.VMEM((CH, D), jnp.float32),
            pltpu.VMEM((CH,), jnp.int32), pltpu.VMEM((CH,), jnp.int32),
            pltpu.VMEM((CH,), jnp.int32), pltpu.VMEM((CH,), jnp.int32),
            pltpu.VMEM((FB, D), jnp.float32), pltpu.VMEM((FB, D), jnp.float32),
            pltpu.VMEM((FB, D), jnp.float32),
            pltpu.VMEM((FB,), jnp.int32),
            pltpu.VMEM_SHARED((AR, D), jnp.float32),
            pltpu.VMEM_SHARED((AR, D), jnp.float32),
            pltpu.VMEM_SHARED((AR, D), jnp.float32),
            pltpu.SemaphoreType.DMA, pltpu.SemaphoreType.DMA,
            pltpu.SemaphoreType.DMA, pltpu.SemaphoreType.DMA,
            pltpu.SemaphoreType.DMA, pltpu.SemaphoreType.DMA,
            pltpu.SemaphoreType.DMA, pltpu.SemaphoreType.DMA,
        ],
    )
    return f(hQK, hV, mapper, batch)


def _sc_edges(rows, cols, attr):
    mesh = plsc.VectorSubcoreMesh(core_axis_name="c", subcore_axis_name="s",
                                  num_cores=NC, num_subcores=NS)
    EPT = EC // NS
    EPW = EC // NW

    def body(row_hbm, col_hbm, attr_hbm, na_hbm,
             row4k, attr4k, hist, hist2, dinv, ident,
             row2k, col2k, attr2k, nbuf, degsp):
        c = lax.axis_index("c")
        s = lax.axis_index("s")
        w = c * NS + s

        iota = lax.iota(jnp.int32, 16)
        zero16 = jnp.zeros((16,), jnp.float32)

        def _z(i, _):
            hist[i, :] = zero16
            return 0
        lax.fori_loop(0, 128, _z, 0)

        ident[...] = iota

        @pl.when(s == 0)
        def _():
            def _z2(i, _):
                for k in range(8):
                    hist2[i, pl.ds(k * 16, 16)] = zero16
                return 0
            lax.fori_loop(0, 16, _z2, 0)
            pltpu.sync_copy(hist2, degsp)

        pltpu.sync_copy(row_hbm.at[pl.ds(s * EPT, EPT)], row4k)
        pltpu.sync_copy(attr_hbm.at[pl.ds(s * EPT, EPT)], attr4k)
        plsc.subcore_barrier()

        def _deg(j, _):
            rvec = row4k[pl.ds(j * 16, 16)]
            avec = attr4k[pl.ds(j * 16, 16)]
            plsc.addupdate_scatter(
                hist, [lax.shift_right_logical(rvec, 4),
                       lax.bitwise_and(rvec, 15)], avec)
            return 0
        lax.fori_loop(0, EPT // 16, _deg, 0)

        def _rp(i, _):
            for k in range(8):
                hist2[i, pl.ds(k * 16, 16)] = hist[i * 8 + k, :]
            return 0
        lax.fori_loop(0, 16, _rp, 0)

        pltpu.sync_copy(hist2, degsp.at[ident], add=True)
        plsc.subcore_barrier()

        pltpu.sync_copy(degsp, hist2)
        half = jnp.full((16,), 0.5, jnp.float32)
        thalf = jnp.full((16,), 1.5, jnp.float32)
        magic = jnp.full((16,), 0x5F3759DF, jnp.int32)

        def _dv(i, _):
            for k in range(8):
                d = hist2[i, pl.ds(k * 16, 16)]
                y = plsc.bitcast(magic - lax.shift_right_logical(
                    plsc.bitcast(d, jnp.int32), 1), jnp.float32)
                hx = half * d
                for _ in range(3):
                    y = y * (thalf - hx * y * y)
                dinv[i, pl.ds(k * 16, 16)] = jnp.where(d > 0.0, y, 0.0)
            return 0
        lax.fori_loop(0, 16, _dv, 0)

        base = w * EPW
        pltpu.sync_copy(row_hbm.at[pl.ds(base, EPW)], row2k)
        pltpu.sync_copy(col_hbm.at[pl.ds(base, EPW)], col2k)
        pltpu.sync_copy(attr_hbm.at[pl.ds(base, EPW)], attr2k)

        def _na(i, _):
            r16 = row2k[pl.ds(i * 16, 16)]
            c16 = col2k[pl.ds(i * 16, 16)]
            a16 = attr2k[pl.ds(i * 16, 16)]
            dr = plsc.load_gather(dinv, [lax.shift_right_logical(r16, 7),
                                         lax.bitwise_and(r16, 127)])
            dc = plsc.load_gather(dinv, [lax.shift_right_logical(c16, 7),
                                         lax.bitwise_and(c16, 127)])
            nbuf[pl.ds(i * 16, 16)] = dr * a16 * dc
            return 0
        lax.fori_loop(0, EPW // 16, _na, 0)

        pltpu.sync_copy(nbuf, na_hbm.at[pl.ds(base, EPW)])

    f = pl.kernel(
        body,
        out_type=jax.ShapeDtypeStruct((EC,), jnp.float32),
        mesh=mesh,
        compiler_params=pltpu.CompilerParams(needs_layout_passes=False),
        scratch_types=[
            pltpu.VMEM((EPT,), jnp.int32), pltpu.VMEM((EPT,), jnp.float32),
            pltpu.VMEM((128, 16), jnp.float32), pltpu.VMEM((16, 128), jnp.float32),
            pltpu.VMEM((16, 128), jnp.float32),
            pltpu.VMEM((16,), jnp.int32),
            pltpu.VMEM((EPW,), jnp.int32), pltpu.VMEM((EPW,), jnp.int32),
            pltpu.VMEM((EPW,), jnp.float32), pltpu.VMEM((EPW,), jnp.float32),
            pltpu.VMEM_SHARED((16, 128), jnp.float32),
        ],
    )
    return f(rows, cols, attr)


def _dense_nodes(x, Wt1, bt1, Wt2, bt2, Wpre1, bpre1, Wpre2, bpre2, Wk, bk, Wv, bv):
    BLK = 2000
    grid = N // BLK

    def _pack2(t):
        lo = lax.bitcast_convert_type(t[:, :64].astype(jnp.bfloat16),
                                      jnp.uint16).astype(jnp.uint32)
        hi = lax.bitcast_convert_type(t[:, 64:].astype(jnp.bfloat16),
                                      jnp.uint16).astype(jnp.uint32)
        return lax.bitcast_convert_type(lo | (hi << 16), jnp.float32)

    def _bdot(a16, b):
        return jnp.dot(a16, b.astype(jnp.bfloat16),
                       preferred_element_type=jnp.float32)

    def body(x_ref, wt1, bt1_, wt2, bt2_, wp1, bp1_, wp2, bp2_, wk, bk_, wv, bv_,
             hqk, hv):
        xb = x_ref[...].astype(jnp.bfloat16)
        h = jnp.maximum(_bdot(xb, wt1[...]) + bt1_[...], 0.0).astype(jnp.bfloat16)
        h = jnp.maximum(_bdot(h, wt2[...]) + bt2_[...], 0.0).astype(jnp.bfloat16)
        q = jnp.maximum(_bdot(h, wp1[...]) + bp1_[...], 0.0).astype(jnp.bfloat16)
        pq = _pack2(_bdot(q, wp2[...]) + bp2_[...])
        pk = _pack2(_bdot(h, wk[...]) + bk_[...])
        hqk[...] = jnp.concatenate([pq, pk], axis=1)
        hv[...] = _bdot(h, wv[...]) + bv_[...]

    wspec = pl.BlockSpec((D, D), lambda i: (0, 0))
    bspec = pl.BlockSpec((1, D), lambda i: (0, 0))
    xspec = pl.BlockSpec((BLK, D), lambda i: (i, 0))
    return pl.pallas_call(
        body,
        grid=(grid,),
        in_specs=[xspec] + [wspec, bspec] * 6,
        out_specs=[xspec, xspec],
        out_shape=[jax.ShapeDtypeStruct((N, D), jnp.float32)] * 2,
    )(x, Wt1, bt1.reshape(1, D), Wt2, bt2.reshape(1, D),
      Wpre1, bpre1.reshape(1, D), Wpre2, bpre2.reshape(1, D),
      Wk, bk.reshape(1, D), Wv, bv.reshape(1, D))


def _post(parts, Wpost1, bpost1, Wpost2, bpost2):
    def body(pr, w1, b1, w2, b2, kq, ksk):
        comb = pr[0] + pr[1]
        sq = comb[0, :P]
        sks = comb[1, :P]
        cnt = comb[2, :P, 0:1]
        t = jnp.maximum(jnp.dot(sq, w1[...], preferred_element_type=jnp.float32) + b1[...], 0.0)
        t = jnp.dot(t, w2[...], preferred_element_type=jnp.float32) + b2[...]
        kq[...] = jnp.maximum(t, 0.0)
        ksk[...] = jnp.maximum(sks / jnp.maximum(cnt, 1.0), 0.0)

    return pl.pallas_call(
        body,
        grid=(1,),
        in_specs=[
            pl.BlockSpec((NC, 3, AR, D), lambda i: (0, 0, 0, 0)),
            pl.BlockSpec((D, D), lambda i: (0, 0)),
            pl.BlockSpec((1, D), lambda i: (0, 0)),
            pl.BlockSpec((D, D), lambda i: (0, 0)),
            pl.BlockSpec((1, D), lambda i: (0, 0)),
        ],
        out_specs=[pl.BlockSpec((P, D), lambda i: (0, 0))] * 2,
        out_shape=[jax.ShapeDtypeStruct((P, D), jnp.float32)] * 2,
    )(parts, Wpost1, bpost1.reshape(1, D), Wpost2, bpost2.reshape(1, D))


def kernel(x, subgraphs_nodes_mapper, subgraphs_batch, subgraphs_batch_row,
           subgraphs_batch_col, coarsen_edge_attr, Wt1, bt1, Wt2, bt2,
           Wpre1, bpre1, Wpre2, bpre2, Wpost1, bpost1, Wpost2, bpost2,
           Wk, bk, Wv, bv):
    mapper = subgraphs_nodes_mapper.astype(jnp.int32)
    batch = subgraphs_batch.astype(jnp.int32)
    erow = subgraphs_batch_row.astype(jnp.int32)
    ecol = subgraphs_batch_col.astype(jnp.int32)

    hQK, hV = _dense_nodes(x, Wt1, bt1, Wt2, bt2, Wpre1, bpre1, Wpre2, bpre2,
                           Wk, bk, Wv, bv)
    norm_attr = _sc_edges(erow, ecol, coarsen_edge_attr)
    kK, V, parts = _sc_main(hQK, hV, mapper, batch)
    kQ, ksK = _post(parts, Wpost1, bpost1, Wpost2, bpost2)
    return (kQ, kK, ksK, V, norm_attr)
